# baseline (speedup 1.0000x reference)
import functools

import jax
import jax.numpy as jnp
from jax import lax
from jax.experimental import pallas as pl
from jax.experimental.pallas import tpu as pltpu
from jax.experimental.pallas import tpu_sc as plsc

_IGNORE = 255
_THRESH = 0.7
_MIN_KEPT = 100000
_K = _MIN_KEPT + 1
_C = 19
_B, _H, _W = 8, 512, 512
_CH = 128
_STEPS = _B * (_H // _CH)
_N = _B * _H * _W

_NW = 32
_LANES = 16
_WSL = _N // _NW
_NB1 = 2048
_NB3 = 1024


def _nll_kernel(pred_ref, tgt_ref, tfloor_ref, keys_ref, stats_ref):
    b = pl.program_id(0)
    h = pl.program_id(1)

    tgt = tgt_ref[0]
    x0 = pred_ref[0, 0]
    m = x0
    for c in range(1, _C):
        m = jnp.maximum(m, pred_ref[0, c])
    z = jnp.exp(x0 - m)
    tl = jnp.where(tgt == 0, x0, 0.0)
    for c in range(1, _C):
        xc = pred_ref[0, c]
        z = z + jnp.exp(xc - m)
        tl = tl + jnp.where(tgt == c, xc, 0.0)
    nll = m + jnp.log(z) - tl
    valid = tgt != _IGNORE
    keys_ref[0] = jnp.where(valid, nll, -1.0)

    tfloor = tfloor_ref[0]
    nv = jnp.sum(valid.astype(jnp.float32))
    asum = jnp.sum(jnp.where(valid, nll, 0.0))
    tfmask = jnp.logical_and(valid, nll >= tfloor)
    tfcnt = jnp.sum(tfmask.astype(jnp.float32))
    tfsum = jnp.sum(jnp.where(tfmask, nll, 0.0))

    @pl.when(jnp.logical_and(b == 0, h == 0))
    def _init():
        stats_ref[0] = 0.0
        stats_ref[1] = 0.0
        stats_ref[2] = 0.0
        stats_ref[3] = 0.0

    stats_ref[0] += nv
    stats_ref[1] += asum
    stats_ref[2] += tfcnt
    stats_ref[3] += tfsum


def _run_nll(pred, target, tfloor):
    return pl.pallas_call(
        _nll_kernel,
        grid=(_B, _H // _CH),
        in_specs=[
            pl.BlockSpec((1, _C, _CH, _W), lambda b, h: (b, 0, h, 0)),
            pl.BlockSpec((1, _CH, _W), lambda b, h: (b, h, 0)),
            pl.BlockSpec(memory_space=pltpu.SMEM),
        ],
        out_specs=[
            pl.BlockSpec((1, _CH, _W), lambda b, h: (b * (_H // _CH) + h, 0, 0)),
            pl.BlockSpec(memory_space=pltpu.SMEM),
        ],
        out_shape=[
            jax.ShapeDtypeStruct((_STEPS, _CH, _W), jnp.float32),
            jax.ShapeDtypeStruct((4,), jnp.float32),
        ],
    )(pred, target, tfloor)


_sc_mesh = plsc.VectorSubcoreMesh(core_axis_name="c", subcore_axis_name="s")
_sc_params = pltpu.CompilerParams(needs_layout_passes=False)


def _worker_id():
    return lax.axis_index("s") * 2 + lax.axis_index("c")


def _scan_rows(keys_hbm, keys_v, wid, vec_body, carry0):
    pltpu.sync_copy(keys_hbm.at[wid], keys_v)

    def row(r, carry):
        for j in range(_W // _LANES):
            v = keys_v[r, pl.ds(j * _LANES, _LANES)]
            ib = lax.bitcast_convert_type(v, jnp.int32)
            carry = vec_body(v, ib, carry)
        return carry

    return lax.fori_loop(0, _CH, row, carry0)


def _memset_zero(ref, nwords):
    zero = jnp.zeros((_LANES,), jnp.int32)

    def body(i, c):
        for j in range(4):
            ref[pl.ds((i * 4 + j) * _LANES, _LANES)] = zero
        return c

    lax.fori_loop(0, nwords // (4 * _LANES), body, jnp.int32(0))


@functools.partial(
    pl.kernel, mesh=_sc_mesh, compiler_params=_sc_params,
    out_type=jax.ShapeDtypeStruct((_NW, _NB1 * _LANES), jnp.int32),
    scratch_types=[
        pltpu.VMEM((_NB1 * _LANES,), jnp.int32),
        pltpu.VMEM((_CH, _W), jnp.float32),
    ],
)
def _sc_scan1(keys_hbm, hist_out, hist_v, keys_v):
    wid = _worker_id()
    _memset_zero(hist_v, _NB1 * _LANES)
    lane = lax.iota(jnp.int32, _LANES)
    ones = jnp.ones((_LANES,), jnp.int32)

    def body(v, ib, c):
        b1 = lax.shift_right_arithmetic(ib, 21)
        idx = lax.shift_left(b1, 4) + lane
        plsc.addupdate_scatter(hist_v, [idx], ones,
                               mask=b1 >= 0)
        return c

    _scan_rows(keys_hbm, keys_v, wid, body, jnp.int32(0))
    pltpu.sync_copy(hist_v, hist_out.at[wid])


@functools.partial(
    pl.kernel, mesh=_sc_mesh, compiler_params=_sc_params,
    out_type=jax.ShapeDtypeStruct((_NW, _NB1 * _LANES), jnp.int32),
    scratch_types=[
        pltpu.VMEM((_NB1 * _LANES,), jnp.int32),
        pltpu.VMEM((_CH, _W), jnp.float32),
        pltpu.VMEM((_LANES,), jnp.int32),
    ],
)
def _sc_scan2(keys_hbm, params_hbm, hist_out, hist_v, keys_v, par_v):
    wid = _worker_id()
    pltpu.sync_copy(params_hbm, par_v)
    _memset_zero(hist_v, _NB1 * _LANES)
    b1 = par_v[...][0]
    lane = lax.iota(jnp.int32, _LANES)
    ones = jnp.ones((_LANES,), jnp.int32)

    def body(v, ib, c):
        sel = lax.shift_right_arithmetic(ib, 21) == b1
        bin2 = jnp.bitwise_and(lax.shift_right_logical(ib, 10), 0x7FF)
        idx = lax.shift_left(bin2, 4) + lane
        plsc.addupdate_scatter(hist_v, [idx], ones, mask=sel)
        return c

    _scan_rows(keys_hbm, keys_v, wid, body, jnp.int32(0))
    pltpu.sync_copy(hist_v, hist_out.at[wid])


@functools.partial(
    pl.kernel, mesh=_sc_mesh, compiler_params=_sc_params,
    out_type=[
        jax.ShapeDtypeStruct((_NW, _NB3 * _LANES), jnp.int32),
        jax.ShapeDtypeStruct((_NW, _LANES), jnp.float32),
        jax.ShapeDtypeStruct((_NW, _LANES), jnp.int32),
    ],
    scratch_types=[
        pltpu.VMEM((_NB3 * _LANES,), jnp.int32),
        pltpu.VMEM((_CH, _W), jnp.float32),
        pltpu.VMEM((_LANES,), jnp.int32),
        pltpu.VMEM((_LANES,), jnp.float32),
        pltpu.VMEM((_LANES,), jnp.int32),
    ],
)
def _sc_scan3(keys_hbm, params_hbm, hist_out, gsum_out, gcnt_out,
              hist_v, keys_v, par_v, gsum_v, gcnt_v):
    wid = _worker_id()
    pltpu.sync_copy(params_hbm, par_v)
    _memset_zero(hist_v, _NB3 * _LANES)
    p22 = par_v[...][0]
    lane = lax.iota(jnp.int32, _LANES)
    ones = jnp.ones((_LANES,), jnp.int32)
    fzero = jnp.zeros((_LANES,), jnp.float32)
    izero = jnp.zeros((_LANES,), jnp.int32)

    def body(v, ib, carry):
        gsum, gcnt = carry
        top22 = lax.shift_right_arithmetic(ib, 10)
        sel = top22 == p22
        bin3 = jnp.bitwise_and(ib, 0x3FF)
        idx = lax.shift_left(bin3, 4) + lane
        plsc.addupdate_scatter(hist_v, [idx], ones, mask=sel)
        above = top22 > p22
        gsum = gsum + jnp.where(above, v, fzero)
        gcnt = gcnt + jnp.where(above, ones, izero)
        return gsum, gcnt

    gsum, gcnt = _scan_rows(keys_hbm, keys_v, wid, body, (fzero, izero))
    gsum_v[...] = gsum
    gcnt_v[...] = gcnt
    pltpu.sync_copy(hist_v, hist_out.at[wid])
    pltpu.sync_copy(gsum_v, gsum_out.at[wid])
    pltpu.sync_copy(gcnt_v, gcnt_out.at[wid])


def _count_ge(hist, binidx, cand):
    return jnp.sum(jnp.where(binidx >= cand, hist, 0))


def _locate_bits(hist, binidx, nbits, k):
    def body(i, v):
        bit = lax.shift_left(jnp.int32(1), nbits - 1 - i)
        cand = v | bit
        cnt = _count_ge(hist, binidx, cand)
        return jnp.where(cnt >= k, cand, v)

    return lax.fori_loop(0, nbits, body, jnp.int32(0))


def _wsum_binidx(hist_ref):
    hsum = jnp.sum(hist_ref[...], axis=0, keepdims=True)
    cols = lax.broadcasted_iota(jnp.int32, hsum.shape, 1)
    return hsum, lax.shift_right_logical(cols, 4)


def _locate1_kernel(hist_ref, out_ref):
    hist, binidx = _wsum_binidx(hist_ref)
    k = jnp.int32(_K)
    b1 = _locate_bits(hist, binidx, 11, k)
    cnt_above = _count_ge(hist, binidx, b1 + 1)
    out_ref[0] = b1
    out_ref[1] = k - cnt_above


def _locate2_kernel(hist_ref, par_ref, out_ref):
    hist, binidx = _wsum_binidx(hist_ref)
    b1 = par_ref[0]
    k2 = par_ref[1]
    b2 = _locate_bits(hist, binidx, 11, k2)
    cnt_above = _count_ge(hist, binidx, b2 + 1)
    out_ref[0] = lax.shift_left(b1, 11) | b2
    out_ref[1] = k2 - cnt_above


def _final_kernel(hist_ref, gsum_ref, gcnt_ref, par_ref, stats_ref, out_ref):
    hist, binidx = _wsum_binidx(hist_ref)
    p22 = par_ref[0]
    k3 = par_ref[1]
    b3 = _locate_bits(hist, binidx, 10, k3)

    histf = hist.astype(jnp.float32)
    inbin_mask = jnp.logical_and(binidx >= b3, hist > 0)
    inbin_cnt = jnp.sum(jnp.where(inbin_mask, histf, 0.0))
    vals = lax.bitcast_convert_type(
        lax.shift_left(p22, 10) | binidx, jnp.float32)
    inbin_sum = jnp.sum(jnp.where(inbin_mask, vals * histf, 0.0))

    gt_cnt = jnp.sum(gcnt_ref[...]).astype(jnp.float32)
    gt_sum = jnp.sum(gsum_ref[...])

    kept_cnt = jnp.maximum(gt_cnt + inbin_cnt, 1.0)
    kept_sum = gt_sum + inbin_sum
    ohem_mean = kept_sum / kept_cnt

    num_valid = stats_ref[0]
    all_sum = stats_ref[1]
    tf_cnt = stats_ref[2]
    tf_sum = stats_ref[3]
    floor_mean = tf_sum / jnp.maximum(tf_cnt, 1.0)
    all_mean = all_sum / jnp.maximum(num_valid, 1.0)

    kept_mean = jnp.where(tf_cnt >= jnp.float32(_K), ohem_mean, floor_mean)
    out_ref[0] = jnp.where(num_valid > jnp.float32(_MIN_KEPT),
                           kept_mean, all_mean)


def _run_locate1(hist):
    return pl.pallas_call(
        _locate1_kernel,
        in_specs=[pl.BlockSpec((_NW, _NB1 * _LANES), lambda: (0, 0))],
        out_specs=pl.BlockSpec(memory_space=pltpu.SMEM),
        out_shape=jax.ShapeDtypeStruct((16,), jnp.int32),
    )(hist)


def _run_locate2(hist, params):
    return pl.pallas_call(
        _locate2_kernel,
        in_specs=[
            pl.BlockSpec((_NW, _NB1 * _LANES), lambda: (0, 0)),
            pl.BlockSpec(memory_space=pltpu.SMEM),
        ],
        out_specs=pl.BlockSpec(memory_space=pltpu.SMEM),
        out_shape=jax.ShapeDtypeStruct((16,), jnp.int32),
    )(hist, params)


def _run_final(hist3, gsum, gcnt, params, stats):
    return pl.pallas_call(
        _final_kernel,
        in_specs=[
            pl.BlockSpec((_NW, _NB3 * _LANES), lambda: (0, 0)),
            pl.BlockSpec((_NW, _LANES), lambda: (0, 0)),
            pl.BlockSpec((_NW, _LANES), lambda: (0, 0)),
            pl.BlockSpec(memory_space=pltpu.SMEM),
            pl.BlockSpec(memory_space=pltpu.SMEM),
        ],
        out_specs=pl.BlockSpec(memory_space=pltpu.SMEM),
        out_shape=jax.ShapeDtypeStruct((1,), jnp.float32),
    )(hist3, gsum, gcnt, params, stats)


@jax.jit
def kernel(pred, target):
    tfloor = -jnp.log(jnp.float32(_THRESH)).reshape(1)
    keys, stats = _run_nll(pred, target, tfloor)

    hist1 = _sc_scan1(keys)
    par1 = _run_locate1(hist1)
    hist2 = _sc_scan2(keys, par1)
    par2 = _run_locate2(hist2, par1)
    hist3, gsum, gcnt = _sc_scan3(keys, par2)
    out = _run_final(hist3, gsum, gcnt, par2, stats)
    return out[0]


# speedup vs baseline: 1.4251x; 1.4251x over previous
import functools

import jax
import jax.numpy as jnp
from jax import lax
from jax.experimental import pallas as pl
from jax.experimental.pallas import tpu as pltpu
from jax.experimental.pallas import tpu_sc as plsc

_IGNORE = 255
_THRESH = 0.7
_MIN_KEPT = 100000
_K = _MIN_KEPT + 1
_C = 19
_B, _H, _W = 8, 512, 512
_CH = 128
_STEPS = _B * (_H // _CH)
_N = _B * _H * _W

_NW = 32
_LANES = 16
_WSL = _N // _NW
_NB1 = 2048
_NB3 = 1024


def _nll_kernel(pred_ref, tgt_ref, tfloor_ref, keys_ref, stats_ref):
    b = pl.program_id(0)
    h = pl.program_id(1)

    tgt = tgt_ref[0]
    x0 = pred_ref[0, 0]
    m = x0
    for c in range(1, _C):
        m = jnp.maximum(m, pred_ref[0, c])
    z = jnp.exp(x0 - m)
    tl = jnp.where(tgt == 0, x0, 0.0)
    for c in range(1, _C):
        xc = pred_ref[0, c]
        z = z + jnp.exp(xc - m)
        tl = tl + jnp.where(tgt == c, xc, 0.0)
    nll = m + jnp.log(z) - tl
    valid = tgt != _IGNORE
    keys_ref[0] = jnp.where(valid, nll, -1.0)

    tfloor = tfloor_ref[0]
    nv = jnp.sum(valid.astype(jnp.float32))
    asum = jnp.sum(jnp.where(valid, nll, 0.0))
    tfmask = jnp.logical_and(valid, nll >= tfloor)
    tfcnt = jnp.sum(tfmask.astype(jnp.float32))
    tfsum = jnp.sum(jnp.where(tfmask, nll, 0.0))

    @pl.when(jnp.logical_and(b == 0, h == 0))
    def _init():
        stats_ref[0] = 0.0
        stats_ref[1] = 0.0
        stats_ref[2] = 0.0
        stats_ref[3] = 0.0

    stats_ref[0] += nv
    stats_ref[1] += asum
    stats_ref[2] += tfcnt
    stats_ref[3] += tfsum


def _run_nll(pred, target, tfloor):
    return pl.pallas_call(
        _nll_kernel,
        grid=(_B, _H // _CH),
        in_specs=[
            pl.BlockSpec((1, _C, _CH, _W), lambda b, h: (b, 0, h, 0)),
            pl.BlockSpec((1, _CH, _W), lambda b, h: (b, h, 0)),
            pl.BlockSpec(memory_space=pltpu.SMEM),
        ],
        out_specs=[
            pl.BlockSpec((1, _CH, _W), lambda b, h: (b * (_H // _CH) + h, 0, 0)),
            pl.BlockSpec(memory_space=pltpu.SMEM),
        ],
        out_shape=[
            jax.ShapeDtypeStruct((_STEPS, _CH, _W), jnp.float32),
            jax.ShapeDtypeStruct((4,), jnp.float32),
        ],
    )(pred, target, tfloor)


_sc_mesh = plsc.VectorSubcoreMesh(core_axis_name="c", subcore_axis_name="s")
_sc_params = pltpu.CompilerParams(needs_layout_passes=False)


def _worker_id():
    return lax.axis_index("s") * 2 + lax.axis_index("c")


def _scan_rows(keys_hbm, keys_v, wid, vec_body, carry0):
    pltpu.sync_copy(keys_hbm.at[wid], keys_v)

    grp = 16
    ng = _CH * _W // (_LANES * grp)
    gpr = _W // (_LANES * grp)

    def group(g, carry):
        r = lax.div(g, jnp.int32(gpr))
        c0 = lax.rem(g, jnp.int32(gpr)) * (_LANES * grp)
        vs = []
        for j in range(grp):
            v = keys_v[r, pl.ds(c0 + j * _LANES, _LANES)]
            vs.append((v, lax.bitcast_convert_type(v, jnp.int32)))
        for v, ib in vs:
            carry = vec_body(v, ib, carry)
        return carry

    return lax.fori_loop(0, ng, group, carry0)


def _memset_zero(ref, nwords):
    zero = jnp.zeros((_LANES,), jnp.int32)

    def body(i, c):
        for j in range(4):
            ref[pl.ds((i * 4 + j) * _LANES, _LANES)] = zero
        return c

    lax.fori_loop(0, nwords // (4 * _LANES), body, jnp.int32(0))


@functools.partial(
    pl.kernel, mesh=_sc_mesh, compiler_params=_sc_params,
    out_type=jax.ShapeDtypeStruct((_NW, _NB1 * _LANES), jnp.int32),
    scratch_types=[
        pltpu.VMEM((_NB1 * _LANES,), jnp.int32),
        pltpu.VMEM((_CH, _W), jnp.float32),
    ],
)
def _sc_scan1(keys_hbm, hist_out, hist_v, keys_v):
    wid = _worker_id()
    _memset_zero(hist_v, _NB1 * _LANES)
    lane = lax.iota(jnp.int32, _LANES)
    ones = jnp.ones((_LANES,), jnp.int32)

    def body(v, ib, c):
        b1 = lax.shift_right_arithmetic(ib, 21)
        idx = lax.shift_left(b1, 4) + lane
        plsc.addupdate_scatter(hist_v, [idx], ones,
                               mask=b1 >= 0)
        return c

    _scan_rows(keys_hbm, keys_v, wid, body, jnp.int32(0))
    pltpu.sync_copy(hist_v, hist_out.at[wid])


@functools.partial(
    pl.kernel, mesh=_sc_mesh, compiler_params=_sc_params,
    out_type=jax.ShapeDtypeStruct((_NW, _NB1 * _LANES), jnp.int32),
    scratch_types=[
        pltpu.VMEM((_NB1 * _LANES,), jnp.int32),
        pltpu.VMEM((_CH, _W), jnp.float32),
        pltpu.VMEM((_LANES,), jnp.int32),
    ],
)
def _sc_scan2(keys_hbm, params_hbm, hist_out, hist_v, keys_v, par_v):
    wid = _worker_id()
    pltpu.sync_copy(params_hbm, par_v)
    _memset_zero(hist_v, _NB1 * _LANES)
    b1 = par_v[...][0]
    lane = lax.iota(jnp.int32, _LANES)
    ones = jnp.ones((_LANES,), jnp.int32)

    def body(v, ib, c):
        sel = lax.shift_right_arithmetic(ib, 21) == b1
        bin2 = jnp.bitwise_and(lax.shift_right_logical(ib, 10), 0x7FF)
        idx = lax.shift_left(bin2, 4) + lane
        plsc.addupdate_scatter(hist_v, [idx], ones, mask=sel)
        return c

    _scan_rows(keys_hbm, keys_v, wid, body, jnp.int32(0))
    pltpu.sync_copy(hist_v, hist_out.at[wid])


@functools.partial(
    pl.kernel, mesh=_sc_mesh, compiler_params=_sc_params,
    out_type=[
        jax.ShapeDtypeStruct((_NW, _NB3 * _LANES), jnp.int32),
        jax.ShapeDtypeStruct((_NW, _LANES), jnp.float32),
        jax.ShapeDtypeStruct((_NW, _LANES), jnp.int32),
    ],
    scratch_types=[
        pltpu.VMEM((_NB3 * _LANES,), jnp.int32),
        pltpu.VMEM((_CH, _W), jnp.float32),
        pltpu.VMEM((_LANES,), jnp.int32),
        pltpu.VMEM((_LANES,), jnp.float32),
        pltpu.VMEM((_LANES,), jnp.int32),
    ],
)
def _sc_scan3(keys_hbm, params_hbm, hist_out, gsum_out, gcnt_out,
              hist_v, keys_v, par_v, gsum_v, gcnt_v):
    wid = _worker_id()
    pltpu.sync_copy(params_hbm, par_v)
    _memset_zero(hist_v, _NB3 * _LANES)
    p22 = par_v[...][0]
    lane = lax.iota(jnp.int32, _LANES)
    ones = jnp.ones((_LANES,), jnp.int32)
    fzero = jnp.zeros((_LANES,), jnp.float32)
    izero = jnp.zeros((_LANES,), jnp.int32)

    def body(v, ib, carry):
        gsum, gcnt = carry
        top22 = lax.shift_right_arithmetic(ib, 10)
        sel = top22 == p22
        bin3 = jnp.bitwise_and(ib, 0x3FF)
        idx = lax.shift_left(bin3, 4) + lane
        plsc.addupdate_scatter(hist_v, [idx], ones, mask=sel)
        above = top22 > p22
        gsum = gsum + jnp.where(above, v, fzero)
        gcnt = gcnt + jnp.where(above, ones, izero)
        return gsum, gcnt

    gsum, gcnt = _scan_rows(keys_hbm, keys_v, wid, body, (fzero, izero))
    gsum_v[...] = gsum
    gcnt_v[...] = gcnt
    pltpu.sync_copy(hist_v, hist_out.at[wid])
    pltpu.sync_copy(gsum_v, gsum_out.at[wid])
    pltpu.sync_copy(gcnt_v, gcnt_out.at[wid])


def _count_ge(hist, binidx, cand):
    return jnp.sum(jnp.where(binidx >= cand, hist, 0))


def _locate_bits(hist, binidx, nbits, k):
    def body(i, v):
        bit = lax.shift_left(jnp.int32(1), nbits - 1 - i)
        cand = v | bit
        cnt = _count_ge(hist, binidx, cand)
        return jnp.where(cnt >= k, cand, v)

    return lax.fori_loop(0, nbits, body, jnp.int32(0))


def _wsum_binidx(hist_ref):
    hsum = jnp.sum(hist_ref[...], axis=0, keepdims=True)
    cols = lax.broadcasted_iota(jnp.int32, hsum.shape, 1)
    return hsum, lax.shift_right_logical(cols, 4)


def _locate1_kernel(hist_ref, out_ref):
    hist, binidx = _wsum_binidx(hist_ref)
    k = jnp.int32(_K)
    b1 = _locate_bits(hist, binidx, 11, k)
    cnt_above = _count_ge(hist, binidx, b1 + 1)
    out_ref[0] = b1
    out_ref[1] = k - cnt_above


def _locate2_kernel(hist_ref, par_ref, out_ref):
    hist, binidx = _wsum_binidx(hist_ref)
    b1 = par_ref[0]
    k2 = par_ref[1]
    b2 = _locate_bits(hist, binidx, 11, k2)
    cnt_above = _count_ge(hist, binidx, b2 + 1)
    out_ref[0] = lax.shift_left(b1, 11) | b2
    out_ref[1] = k2 - cnt_above


def _final_kernel(hist_ref, gsum_ref, gcnt_ref, par_ref, stats_ref, out_ref):
    hist, binidx = _wsum_binidx(hist_ref)
    p22 = par_ref[0]
    k3 = par_ref[1]
    b3 = _locate_bits(hist, binidx, 10, k3)

    histf = hist.astype(jnp.float32)
    inbin_mask = jnp.logical_and(binidx >= b3, hist > 0)
    inbin_cnt = jnp.sum(jnp.where(inbin_mask, histf, 0.0))
    vals = lax.bitcast_convert_type(
        lax.shift_left(p22, 10) | binidx, jnp.float32)
    inbin_sum = jnp.sum(jnp.where(inbin_mask, vals * histf, 0.0))

    gt_cnt = jnp.sum(gcnt_ref[...]).astype(jnp.float32)
    gt_sum = jnp.sum(gsum_ref[...])

    kept_cnt = jnp.maximum(gt_cnt + inbin_cnt, 1.0)
    kept_sum = gt_sum + inbin_sum
    ohem_mean = kept_sum / kept_cnt

    num_valid = stats_ref[0]
    all_sum = stats_ref[1]
    tf_cnt = stats_ref[2]
    tf_sum = stats_ref[3]
    floor_mean = tf_sum / jnp.maximum(tf_cnt, 1.0)
    all_mean = all_sum / jnp.maximum(num_valid, 1.0)

    kept_mean = jnp.where(tf_cnt >= jnp.float32(_K), ohem_mean, floor_mean)
    out_ref[0] = jnp.where(num_valid > jnp.float32(_MIN_KEPT),
                           kept_mean, all_mean)


def _run_locate1(hist):
    return pl.pallas_call(
        _locate1_kernel,
        in_specs=[pl.BlockSpec((_NW, _NB1 * _LANES), lambda: (0, 0))],
        out_specs=pl.BlockSpec(memory_space=pltpu.SMEM),
        out_shape=jax.ShapeDtypeStruct((16,), jnp.int32),
    )(hist)


def _run_locate2(hist, params):
    return pl.pallas_call(
        _locate2_kernel,
        in_specs=[
            pl.BlockSpec((_NW, _NB1 * _LANES), lambda: (0, 0)),
            pl.BlockSpec(memory_space=pltpu.SMEM),
        ],
        out_specs=pl.BlockSpec(memory_space=pltpu.SMEM),
        out_shape=jax.ShapeDtypeStruct((16,), jnp.int32),
    )(hist, params)


def _run_final(hist3, gsum, gcnt, params, stats):
    return pl.pallas_call(
        _final_kernel,
        in_specs=[
            pl.BlockSpec((_NW, _NB3 * _LANES), lambda: (0, 0)),
            pl.BlockSpec((_NW, _LANES), lambda: (0, 0)),
            pl.BlockSpec((_NW, _LANES), lambda: (0, 0)),
            pl.BlockSpec(memory_space=pltpu.SMEM),
            pl.BlockSpec(memory_space=pltpu.SMEM),
        ],
        out_specs=pl.BlockSpec(memory_space=pltpu.SMEM),
        out_shape=jax.ShapeDtypeStruct((1,), jnp.float32),
    )(hist3, gsum, gcnt, params, stats)


@jax.jit
def kernel(pred, target):
    tfloor = -jnp.log(jnp.float32(_THRESH)).reshape(1)
    keys, stats = _run_nll(pred, target, tfloor)

    hist1 = _sc_scan1(keys)
    par1 = _run_locate1(hist1)
    hist2 = _sc_scan2(keys, par1)
    par2 = _run_locate2(hist2, par1)
    hist3, gsum, gcnt = _sc_scan3(keys, par2)
    out = _run_final(hist3, gsum, gcnt, par2, stats)
    return out[0]


# speedup vs baseline: 1.5912x; 1.1166x over previous
import functools

import jax
import jax.numpy as jnp
from jax import lax
from jax.experimental import pallas as pl
from jax.experimental.pallas import tpu as pltpu
from jax.experimental.pallas import tpu_sc as plsc

_IGNORE = 255
_THRESH = 0.7
_MIN_KEPT = 100000
_K = _MIN_KEPT + 1
_C = 19
_B, _H, _W = 8, 512, 512
_CH = 128
_STEPS = _B * (_H // _CH)
_N = _B * _H * _W

_NW = 32
_LANES = 16
_WSL = _N // _NW
_NB1 = 2048
_NB3 = 1024


def _nll_kernel(pred_ref, tgt_ref, tfloor_ref, keys_ref, stats_ref):
    b = pl.program_id(0)
    h = pl.program_id(1)
    tfloor = tfloor_ref[0]

    sub = 8
    nv = jnp.float32(0.0)
    asum = jnp.float32(0.0)
    tfcnt = jnp.float32(0.0)
    tfsum = jnp.float32(0.0)
    for s in range(_CH // sub):
        rows = pl.ds(s * sub, sub)
        tgt = tgt_ref[0, rows]
        xs = [pred_ref[0, c, rows] for c in range(_C)]
        m = xs[0]
        for c in range(1, _C):
            m = jnp.maximum(m, xs[c])
        z = jnp.exp(xs[0] - m)
        tl = jnp.where(tgt == 0, xs[0], 0.0)
        for c in range(1, _C):
            z = z + jnp.exp(xs[c] - m)
            tl = tl + jnp.where(tgt == c, xs[c], 0.0)
        nll = m + jnp.log(z) - tl
        valid = tgt != _IGNORE
        keys_ref[0, rows] = jnp.where(valid, nll, -1.0)

        vmask = jnp.where(valid, nll, 0.0)
        tfmask = jnp.logical_and(valid, nll >= tfloor)
        nv += jnp.sum(valid.astype(jnp.float32))
        asum += jnp.sum(vmask)
        tfcnt += jnp.sum(tfmask.astype(jnp.float32))
        tfsum += jnp.sum(jnp.where(tfmask, nll, 0.0))

    @pl.when(jnp.logical_and(b == 0, h == 0))
    def _init():
        stats_ref[0] = 0.0
        stats_ref[1] = 0.0
        stats_ref[2] = 0.0
        stats_ref[3] = 0.0

    stats_ref[0] += nv
    stats_ref[1] += asum
    stats_ref[2] += tfcnt
    stats_ref[3] += tfsum


def _run_nll(pred, target, tfloor):
    return pl.pallas_call(
        _nll_kernel,
        grid=(_B, _H // _CH),
        in_specs=[
            pl.BlockSpec((1, _C, _CH, _W), lambda b, h: (b, 0, h, 0)),
            pl.BlockSpec((1, _CH, _W), lambda b, h: (b, h, 0)),
            pl.BlockSpec(memory_space=pltpu.SMEM),
        ],
        out_specs=[
            pl.BlockSpec((1, _CH, _W), lambda b, h: (b * (_H // _CH) + h, 0, 0)),
            pl.BlockSpec(memory_space=pltpu.SMEM),
        ],
        out_shape=[
            jax.ShapeDtypeStruct((_STEPS, _CH, _W), jnp.float32),
            jax.ShapeDtypeStruct((4,), jnp.float32),
        ],
    )(pred, target, tfloor)


_sc_mesh = plsc.VectorSubcoreMesh(core_axis_name="c", subcore_axis_name="s")
_sc_params = pltpu.CompilerParams(needs_layout_passes=False)


def _worker_id():
    return lax.axis_index("s") * 2 + lax.axis_index("c")


def _scan_rows(keys_hbm, keys_v, wid, vec_body, carry0):
    pltpu.sync_copy(keys_hbm.at[wid], keys_v)

    grp = 16
    ng = _CH * _W // (_LANES * grp)
    gpr = _W // (_LANES * grp)

    def group(g, carry):
        r = lax.div(g, jnp.int32(gpr))
        c0 = lax.rem(g, jnp.int32(gpr)) * (_LANES * grp)
        vs = []
        for j in range(grp):
            v = keys_v[r, pl.ds(c0 + j * _LANES, _LANES)]
            vs.append((v, lax.bitcast_convert_type(v, jnp.int32)))
        for v, ib in vs:
            carry = vec_body(v, ib, carry)
        return carry

    return lax.fori_loop(0, ng, group, carry0)


def _memset_zero(ref, nwords):
    zero = jnp.zeros((_LANES,), jnp.int32)

    def body(i, c):
        for j in range(4):
            ref[pl.ds((i * 4 + j) * _LANES, _LANES)] = zero
        return c

    lax.fori_loop(0, nwords // (4 * _LANES), body, jnp.int32(0))


@functools.partial(
    pl.kernel, mesh=_sc_mesh, compiler_params=_sc_params,
    out_type=jax.ShapeDtypeStruct((_NW, _NB1 * _LANES), jnp.int32),
    scratch_types=[
        pltpu.VMEM((_NB1 * _LANES,), jnp.int32),
        pltpu.VMEM((_CH, _W), jnp.float32),
    ],
)
def _sc_scan1(keys_hbm, hist_out, hist_v, keys_v):
    wid = _worker_id()
    _memset_zero(hist_v, _NB1 * _LANES)
    lane = lax.iota(jnp.int32, _LANES)
    ones = jnp.ones((_LANES,), jnp.int32)

    def body(v, ib, c):
        b1 = lax.shift_right_arithmetic(ib, 21)
        idx = lax.shift_left(b1, 4) + lane
        plsc.addupdate_scatter(hist_v, [idx], ones,
                               mask=b1 >= 0)
        return c

    _scan_rows(keys_hbm, keys_v, wid, body, jnp.int32(0))
    pltpu.sync_copy(hist_v, hist_out.at[wid])


@functools.partial(
    pl.kernel, mesh=_sc_mesh, compiler_params=_sc_params,
    out_type=jax.ShapeDtypeStruct((_NW, _NB1 * _LANES), jnp.int32),
    scratch_types=[
        pltpu.VMEM((_NB1 * _LANES,), jnp.int32),
        pltpu.VMEM((_CH, _W), jnp.float32),
        pltpu.VMEM((_LANES,), jnp.int32),
    ],
)
def _sc_scan2(keys_hbm, params_hbm, hist_out, hist_v, keys_v, par_v):
    wid = _worker_id()
    pltpu.sync_copy(params_hbm, par_v)
    _memset_zero(hist_v, _NB1 * _LANES)
    b1 = par_v[...][0]
    lane = lax.iota(jnp.int32, _LANES)
    ones = jnp.ones((_LANES,), jnp.int32)

    def body(v, ib, c):
        sel = lax.shift_right_arithmetic(ib, 21) == b1
        bin2 = jnp.bitwise_and(lax.shift_right_logical(ib, 10), 0x7FF)
        idx = lax.shift_left(bin2, 4) + lane
        plsc.addupdate_scatter(hist_v, [idx], ones, mask=sel)
        return c

    _scan_rows(keys_hbm, keys_v, wid, body, jnp.int32(0))
    pltpu.sync_copy(hist_v, hist_out.at[wid])


@functools.partial(
    pl.kernel, mesh=_sc_mesh, compiler_params=_sc_params,
    out_type=[
        jax.ShapeDtypeStruct((_NW, _NB3 * _LANES), jnp.int32),
        jax.ShapeDtypeStruct((_NW, _LANES), jnp.float32),
        jax.ShapeDtypeStruct((_NW, _LANES), jnp.int32),
    ],
    scratch_types=[
        pltpu.VMEM((_NB3 * _LANES,), jnp.int32),
        pltpu.VMEM((_CH, _W), jnp.float32),
        pltpu.VMEM((_LANES,), jnp.int32),
        pltpu.VMEM((_LANES,), jnp.float32),
        pltpu.VMEM((_LANES,), jnp.int32),
    ],
)
def _sc_scan3(keys_hbm, params_hbm, hist_out, gsum_out, gcnt_out,
              hist_v, keys_v, par_v, gsum_v, gcnt_v):
    wid = _worker_id()
    pltpu.sync_copy(params_hbm, par_v)
    _memset_zero(hist_v, _NB3 * _LANES)
    p22 = par_v[...][0]
    lane = lax.iota(jnp.int32, _LANES)
    ones = jnp.ones((_LANES,), jnp.int32)
    fzero = jnp.zeros((_LANES,), jnp.float32)
    izero = jnp.zeros((_LANES,), jnp.int32)

    def body(v, ib, carry):
        gsum, gcnt = carry
        top22 = lax.shift_right_arithmetic(ib, 10)
        sel = top22 == p22
        bin3 = jnp.bitwise_and(ib, 0x3FF)
        idx = lax.shift_left(bin3, 4) + lane
        plsc.addupdate_scatter(hist_v, [idx], ones, mask=sel)
        above = top22 > p22
        gsum = gsum + jnp.where(above, v, fzero)
        gcnt = gcnt + jnp.where(above, ones, izero)
        return gsum, gcnt

    gsum, gcnt = _scan_rows(keys_hbm, keys_v, wid, body, (fzero, izero))
    gsum_v[...] = gsum
    gcnt_v[...] = gcnt
    pltpu.sync_copy(hist_v, hist_out.at[wid])
    pltpu.sync_copy(gsum_v, gsum_out.at[wid])
    pltpu.sync_copy(gcnt_v, gcnt_out.at[wid])


def _count_ge(hist, binidx, cand):
    return jnp.sum(jnp.where(binidx >= cand, hist, 0))


def _locate_bits(hist, binidx, nbits, k):
    def body(i, v):
        bit = lax.shift_left(jnp.int32(1), nbits - 1 - i)
        cand = v | bit
        cnt = _count_ge(hist, binidx, cand)
        return jnp.where(cnt >= k, cand, v)

    return lax.fori_loop(0, nbits, body, jnp.int32(0))


def _wsum_binidx(hist_ref):
    hsum = jnp.sum(hist_ref[...], axis=0, keepdims=True)
    cols = lax.broadcasted_iota(jnp.int32, hsum.shape, 1)
    return hsum, lax.shift_right_logical(cols, 4)


def _locate1_kernel(hist_ref, out_ref):
    hist, binidx = _wsum_binidx(hist_ref)
    k = jnp.int32(_K)
    b1 = _locate_bits(hist, binidx, 11, k)
    cnt_above = _count_ge(hist, binidx, b1 + 1)
    out_ref[0] = b1
    out_ref[1] = k - cnt_above


def _locate2_kernel(hist_ref, par_ref, out_ref):
    hist, binidx = _wsum_binidx(hist_ref)
    b1 = par_ref[0]
    k2 = par_ref[1]
    b2 = _locate_bits(hist, binidx, 11, k2)
    cnt_above = _count_ge(hist, binidx, b2 + 1)
    out_ref[0] = lax.shift_left(b1, 11) | b2
    out_ref[1] = k2 - cnt_above


def _final_kernel(hist_ref, gsum_ref, gcnt_ref, par_ref, stats_ref, out_ref):
    hist, binidx = _wsum_binidx(hist_ref)
    p22 = par_ref[0]
    k3 = par_ref[1]
    b3 = _locate_bits(hist, binidx, 10, k3)

    histf = hist.astype(jnp.float32)
    inbin_mask = jnp.logical_and(binidx >= b3, hist > 0)
    inbin_cnt = jnp.sum(jnp.where(inbin_mask, histf, 0.0))
    vals = lax.bitcast_convert_type(
        lax.shift_left(p22, 10) | binidx, jnp.float32)
    inbin_sum = jnp.sum(jnp.where(inbin_mask, vals * histf, 0.0))

    gt_cnt = jnp.sum(gcnt_ref[...]).astype(jnp.float32)
    gt_sum = jnp.sum(gsum_ref[...])

    kept_cnt = jnp.maximum(gt_cnt + inbin_cnt, 1.0)
    kept_sum = gt_sum + inbin_sum
    ohem_mean = kept_sum / kept_cnt

    num_valid = stats_ref[0]
    all_sum = stats_ref[1]
    tf_cnt = stats_ref[2]
    tf_sum = stats_ref[3]
    floor_mean = tf_sum / jnp.maximum(tf_cnt, 1.0)
    all_mean = all_sum / jnp.maximum(num_valid, 1.0)

    kept_mean = jnp.where(tf_cnt >= jnp.float32(_K), ohem_mean, floor_mean)
    out_ref[0] = jnp.where(num_valid > jnp.float32(_MIN_KEPT),
                           kept_mean, all_mean)


def _run_locate1(hist):
    return pl.pallas_call(
        _locate1_kernel,
        in_specs=[pl.BlockSpec((_NW, _NB1 * _LANES), lambda: (0, 0))],
        out_specs=pl.BlockSpec(memory_space=pltpu.SMEM),
        out_shape=jax.ShapeDtypeStruct((16,), jnp.int32),
    )(hist)


def _run_locate2(hist, params):
    return pl.pallas_call(
        _locate2_kernel,
        in_specs=[
            pl.BlockSpec((_NW, _NB1 * _LANES), lambda: (0, 0)),
            pl.BlockSpec(memory_space=pltpu.SMEM),
        ],
        out_specs=pl.BlockSpec(memory_space=pltpu.SMEM),
        out_shape=jax.ShapeDtypeStruct((16,), jnp.int32),
    )(hist, params)


def _run_final(hist3, gsum, gcnt, params, stats):
    return pl.pallas_call(
        _final_kernel,
        in_specs=[
            pl.BlockSpec((_NW, _NB3 * _LANES), lambda: (0, 0)),
            pl.BlockSpec((_NW, _LANES), lambda: (0, 0)),
            pl.BlockSpec((_NW, _LANES), lambda: (0, 0)),
            pl.BlockSpec(memory_space=pltpu.SMEM),
            pl.BlockSpec(memory_space=pltpu.SMEM),
        ],
        out_specs=pl.BlockSpec(memory_space=pltpu.SMEM),
        out_shape=jax.ShapeDtypeStruct((1,), jnp.float32),
    )(hist3, gsum, gcnt, params, stats)


@jax.jit
def kernel(pred, target):
    tfloor = -jnp.log(jnp.float32(_THRESH)).reshape(1)
    keys, stats = _run_nll(pred, target, tfloor)

    hist1 = _sc_scan1(keys)
    par1 = _run_locate1(hist1)
    hist2 = _sc_scan2(keys, par1)
    par2 = _run_locate2(hist2, par1)
    hist3, gsum, gcnt = _sc_scan3(keys, par2)
    out = _run_final(hist3, gsum, gcnt, par2, stats)
    return out[0]
